# baseline (speedup 1.0000x reference)
import functools

import jax
import jax.numpy as jnp
from jax import lax
from jax.experimental import pallas as pl
from jax.experimental.pallas import tpu as pltpu
from jax.experimental.pallas import tpu_sc as plsc

N_ENT = 100000
N_USR = 50000
N_FAC = 8
D = 128

NC = 2
NS = 16
L = 16

C = 12800
CT = C + 128
KB = 2048
NB = 32
ZR = 16

_CPARAMS = pltpu.CompilerParams(needs_layout_passes=False)
_MESH = dict(core_axis_name="c", subcore_axis_name="s")

_BLK = 1024


def _cdiv(a, b):
    return (a + b - 1) // b


@functools.lru_cache(maxsize=None)
def _build_segsum(n_dst, e_pad, mode):
    n_chunks = _cdiv(n_dst, C)
    out_rows = n_chunks * C
    P = _cdiv(n_chunks, NC)
    Et = e_pad // NS
    nblk = Et // KB
    assert Et % KB == 0
    slice_rows = CT // NS
    wrows = C // NS

    scratch = [
        pltpu.VMEM_SHARED((CT, D), jnp.float32),
        pltpu.VMEM((KB,), jnp.int32),
        pltpu.VMEM((KB,), jnp.int32),
        pltpu.VMEM((KB + 4 * L,), jnp.int32),
        pltpu.VMEM((KB + 4 * L,), jnp.int32),
        pltpu.VMEM((2 * NB, D), jnp.float32),
        pltpu.VMEM((ZR, D), jnp.float32),
        pltpu.SemaphoreType.DMA,
    ]
    if mode == "scalar":
        scratch += [pltpu.VMEM((KB,), jnp.float32),
                    pltpu.VMEM((KB + 4 * L,), jnp.float32)]
    elif mode == "rel":
        scratch += [pltpu.VMEM((KB,), jnp.int32),
                    pltpu.VMEM((KB + 4 * L,), jnp.int32),
                    pltpu.VMEM((32 * D,), jnp.float32)]

    def body(*refs):
        if mode == "plain":
            (table, idx_dst_h, idx_src_h, out_h,
             spmem, dst_v, src_v, acc_dst, acc_src, rows, zrow, sem) = refs
            w_v = acc_w = rel_buf = None
        elif mode == "scalar":
            (table, idx_dst_h, idx_src_h, w_h, out_h,
             spmem, dst_v, src_v, acc_dst, acc_src, rows, zrow, sem,
             w_v, acc_w) = refs
            rel_buf = None
        else:
            (table, idx_dst_h, idx_src_h, w_h, rel_h, out_h,
             spmem, dst_v, src_v, acc_dst, acc_src, rows, zrow, sem,
             w_v, acc_w, rel_buf) = refs

        s = lax.axis_index("s")
        c = lax.axis_index("c")
        zero16 = jnp.zeros((L,), jnp.float32)

        for rz in range(ZR):
            for j in range(D // L):
                zrow[rz, pl.ds(j * L, L)] = zero16

        def zinit(i, carry):
            acc_src[pl.ds(i * L, L)] = jnp.zeros((L,), jnp.int32)
            return carry

        lax.fori_loop(0, (KB + 4 * L) // L, zinit, jnp.int32(0))
        if mode == "rel":
            pltpu.sync_copy(rel_h, rel_buf)

        def scale_batch(slot, off):
            if mode == "scalar":
                for half in range(NB // L):
                    wv = acc_w[pl.ds(off + half * L, L)]
                    for rr in range(L):
                        w = wv[rr]
                        row = slot * NB + half * L + rr
                        for j in range(D // L):
                            rows[row, pl.ds(j * L, L)] = (
                                rows[row, pl.ds(j * L, L)] * w)
            elif mode == "rel":
                for half in range(NB // L):
                    etv = acc_w[pl.ds(off + half * L, L)]
                    for rr in range(L):
                        rb = etv[rr] * D
                        row = slot * NB + half * L + rr
                        for j in range(D // L):
                            rows[row, pl.ds(j * L, L)] = (
                                rows[row, pl.ds(j * L, L)]
                                * rel_buf[pl.ds(rb + j * L, L)])

        def fire(b, slot):
            sv0 = acc_src[pl.ds(b * NB, L)]
            sv1 = acc_src[pl.ds(b * NB + L, L)]
            pltpu.async_copy(table.at[sv0],
                             rows.at[pl.ds(slot * NB, L)], sem)
            pltpu.async_copy(table.at[sv1],
                             rows.at[pl.ds(slot * NB + L, L)], sem)

        def wait_g(slot):
            for h in range(2):
                pltpu.make_async_copy(
                    table.at[pl.ds(0, L)],
                    rows.at[pl.ds(slot * NB + h * L, L)], sem,
                ).wait()

        def chunk_body(p, carry):
            chunk = p * NC + c

            @pl.when(chunk < n_chunks)
            def _():
                lo = chunk * C
                hi = lo + C
                zbase = s * slice_rows
                for kk in range(slice_rows // ZR):
                    pltpu.sync_copy(zrow, spmem.at[pl.ds(zbase + kk * ZR, ZR)])
                rem = slice_rows % ZR
                if rem:
                    pltpu.sync_copy(
                        zrow.at[pl.ds(0, rem)],
                        spmem.at[pl.ds(zbase + (slice_rows // ZR) * ZR, rem)])
                plsc.subcore_barrier()

                def block_body(b, carry2):
                    base = s * Et + b * KB
                    pltpu.sync_copy(idx_dst_h.at[pl.ds(base, KB)], dst_v)
                    pltpu.sync_copy(idx_src_h.at[pl.ds(base, KB)], src_v)
                    if mode != "plain":
                        pltpu.sync_copy(w_h.at[pl.ds(base, KB)], w_v)

                    def cvec(k2, n):
                        dv = dst_v[pl.ds(k2 * L, L)]
                        m = (dv >= lo) & (dv < hi)
                        mi = m.astype(jnp.int32)
                        pos = n + plsc.cumsum(mi) - 1
                        plsc.store_scatter(acc_dst, [pos], dv - lo, mask=m)
                        sv = src_v[pl.ds(k2 * L, L)]
                        plsc.store_scatter(acc_src, [pos], sv, mask=m)
                        if mode != "plain":
                            wv = w_v[pl.ds(k2 * L, L)]
                            plsc.store_scatter(acc_w, [pos], wv, mask=m)
                        return n + jnp.sum(mi)

                    n = lax.fori_loop(0, KB // L, cvec, jnp.int32(0))

                    iota = lax.iota(jnp.int32, L)
                    trash = jnp.int32(C) + iota
                    dummy_src = iota * 16 + s * 37
                    for t in range(4):
                        acc_dst[pl.ds(n + t * L, L)] = trash
                        acc_src[pl.ds(n + t * L, L)] = dummy_src
                        if mode != "plain":
                            acc_w[pl.ds(n + t * L, L)] = (
                                jnp.zeros((L,), acc_w.dtype))

                    nb = (n + NB - 1) // NB
                    nbp = (nb + 1) // 2
                    fire(jnp.int32(0), 0)
                    fire(jnp.int32(1), 1)

                    def drain(i, carry3):
                        for slot in range(2):
                            bcur = 2 * i + slot
                            off = bcur * NB
                            wait_g(slot)
                            scale_batch(slot, off)
                            dv0 = acc_dst[pl.ds(off, L)]
                            dv1 = acc_dst[pl.ds(off + L, L)]
                            pltpu.sync_copy(rows.at[pl.ds(slot * NB, L)],
                                            spmem.at[dv0], add=True)
                            pltpu.sync_copy(rows.at[pl.ds(slot * NB + L, L)],
                                            spmem.at[dv1], add=True)
                            fire(bcur + 2, slot)
                        return carry3

                    lax.fori_loop(0, nbp, drain, jnp.int32(0))
                    wait_g(0)
                    wait_g(1)
                    return carry2

                lax.fori_loop(0, nblk, block_body, jnp.int32(0))
                plsc.subcore_barrier()
                pltpu.sync_copy(
                    spmem.at[pl.ds(s * wrows, wrows)],
                    out_h.at[pl.ds(chunk * C + s * wrows, wrows)])
                plsc.subcore_barrier()

            return carry

        lax.fori_loop(0, P, chunk_body, jnp.int32(0))

    return pl.kernel(
        body,
        out_type=jax.ShapeDtypeStruct((out_rows, D), jnp.float32),
        mesh=plsc.VectorSubcoreMesh(**_MESH),
        scratch_types=scratch,
        compiler_params=_CPARAMS,
    )


@functools.lru_cache(maxsize=None)
def _build_counts(n_dst_pad, e1_pad, e2_pad):
    buf = n_dst_pad + 8 * L
    assert buf % (NS * 8) == 0
    per_tile = buf // NS
    wr = n_dst_pad // NS

    def body(idx1_h, idx2_h, out_h, cntA, cntB, idx_v, ones_v, zv):
        s = lax.axis_index("s")
        c = lax.axis_index("c")

        def fill(i, carry):
            zv[pl.ds(i * L, L)] = jnp.zeros((L,), jnp.float32)
            ones_v[pl.ds(i * L, L)] = jnp.ones((L,), jnp.float32)
            return carry

        lax.fori_loop(0, KB // L, fill, jnp.int32(0))

        zbase = s * per_tile
        nz = per_tile // KB
        for cnt in (cntA, cntB):
            for kk in range(nz):
                pltpu.sync_copy(zv, cnt.at[pl.ds(zbase + kk * KB, KB)])
            rem = per_tile % KB
            if rem:
                pltpu.sync_copy(zv.at[pl.ds(0, rem)],
                                cnt.at[pl.ds(zbase + nz * KB, rem)])
        plsc.subcore_barrier()

        for idx_h, cnt, e_pad in ((idx1_h, cntA, e1_pad),
                                  (idx2_h, cntB, e2_pad)):
            Et = e_pad // (NC * NS)
            nblk = Et // KB
            assert Et % KB == 0

            def block_body(b, carry, idx_h=idx_h, cnt=cnt, Et=Et):
                base = (c * NS + s) * Et + b * KB
                pltpu.sync_copy(idx_h.at[pl.ds(base, KB)], idx_v)
                pltpu.sync_copy(ones_v, cnt.at[idx_v], add=True)
                return carry

            lax.fori_loop(0, nblk, block_body, jnp.int32(0))
        plsc.subcore_barrier()
        pltpu.sync_copy(cntA.at[pl.ds(s * wr, wr)],
                        out_h.at[c, pl.ds(s * wr, wr)])
        pltpu.sync_copy(cntB.at[pl.ds(s * wr, wr)],
                        out_h.at[c, pl.ds(n_dst_pad + s * wr, wr)])

    return pl.kernel(
        body,
        out_type=jax.ShapeDtypeStruct((NC, 2 * n_dst_pad), jnp.float32),
        mesh=plsc.VectorSubcoreMesh(**_MESH),
        scratch_types=[
            pltpu.VMEM_SHARED((buf,), jnp.float32),
            pltpu.VMEM_SHARED((buf,), jnp.float32),
            pltpu.VMEM((KB,), jnp.int32),
            pltpu.VMEM((KB,), jnp.float32),
            pltpu.VMEM((KB,), jnp.float32),
        ],
        compiler_params=_CPARAMS,
    )


def _finalize_body(sum_ref, cnt_ref, res_ref, emb_out_ref, res_out_ref):
    sval = sum_ref[...]
    cval = cnt_ref[...]
    mean = sval / jnp.maximum(cval, 1.0)
    nrm = jnp.sqrt(jnp.sum(mean * mean, axis=-1, keepdims=True))
    emb = mean / jnp.maximum(nrm, 1e-12)
    emb_out_ref[...] = emb
    res_out_ref[...] = res_ref[...] + emb


def _finalize(seg_sum, cnt, res, n_rows):
    pad = (-n_rows) % _BLK
    seg_sum = seg_sum[:n_rows]
    if pad:
        seg_sum = jnp.pad(seg_sum, ((0, pad), (0, 0)))
        cnt = jnp.pad(cnt, ((0, pad), (0, 0)))
        res = jnp.pad(res, ((0, pad), (0, 0)))
    n_pad = n_rows + pad
    emb, res_out = pl.pallas_call(
        _finalize_body,
        grid=(n_pad // _BLK,),
        in_specs=[
            pl.BlockSpec((_BLK, D), lambda i: (i, 0)),
            pl.BlockSpec((_BLK, 1), lambda i: (i, 0)),
            pl.BlockSpec((_BLK, D), lambda i: (i, 0)),
        ],
        out_specs=[
            pl.BlockSpec((_BLK, D), lambda i: (i, 0)),
            pl.BlockSpec((_BLK, D), lambda i: (i, 0)),
        ],
        out_shape=[
            jax.ShapeDtypeStruct((n_pad, D), jnp.float32),
            jax.ShapeDtypeStruct((n_pad, D), jnp.float32),
        ],
    )(seg_sum, cnt, res)
    return emb[:n_rows], res_out[:n_rows]


def _user_finalize(user_agg, usr, intent, res):
    score = jax.nn.softmax(usr @ intent.T, axis=1)
    usr_agg = (score @ intent) * user_agg + user_agg
    ones = jnp.ones((usr_agg.shape[0], 1), jnp.float32)
    return _finalize(usr_agg, ones, res, usr_agg.shape[0])


def _pad_to(x, e_pad, fill):
    if x.shape[0] == e_pad:
        return x
    return jnp.concatenate([x, fill])


def _pad_edges(dst, src, w, n_dst_pad, n_src):
    e = dst.shape[0]
    step = NC * NS * KB
    e_pad = _cdiv(e, step) * step
    npad = e_pad - e
    ar = lax.iota(jnp.int32, npad)
    dstp = _pad_to(dst.astype(jnp.int32), e_pad,
                   n_dst_pad + (ar % (8 * L)))
    srcp = _pad_to(src.astype(jnp.int32), e_pad, ar % n_src)
    wp = None
    if w is not None:
        fill = (jnp.zeros((npad,), w.dtype) if w.dtype == jnp.float32
                else (ar % 32).astype(w.dtype))
        wp = _pad_to(w, e_pad, fill)
    return dstp, srcp, wp, e_pad


def kernel(user_emb, entity_emb, rel_weight, rel_intent_proj, hist_intent,
           ui_vals, edge_index, edge_type, ui_rows, ui_cols,
           adj_user_idx, adj_item_idx):
    relation_emb = rel_weight
    rel_int = rel_intent_proj @ relation_emb
    hist_int = hist_intent
    head, tail = edge_index[0], edge_index[1]

    n_ent_chunks = _cdiv(N_ENT, C)
    n_usr_chunks = _cdiv(N_USR, C)
    ent_pad = n_ent_chunks * C
    usr_pad = n_usr_chunks * C

    ui_r, ui_c, ui_w, e_ui = _pad_edges(ui_rows, ui_cols, ui_vals,
                                        usr_pad, N_ENT)
    adj_i, adj_u, _, e_adj = _pad_edges(adj_item_idx, adj_user_idx, None,
                                        ent_pad, N_USR)
    ehead, etail, etype, e_edge = _pad_edges(head, tail, edge_type,
                                             ent_pad, N_ENT)
    rel_flat = relation_emb.reshape((32 * D,))

    cnt2 = _build_counts(ent_pad, e_adj, e_edge)(adj_i, ehead)
    cnts = cnt2[0] + cnt2[1]
    cnt_adj = cnts[:N_ENT, None]
    cnt_edge = cnts[ent_pad:ent_pad + N_ENT, None]

    ui_k = _build_segsum(N_USR, e_ui, "scalar")
    adj_k = _build_segsum(N_ENT, e_adj, "plain")
    edge_k = _build_segsum(N_ENT, e_edge, "rel")

    def ui_agg(ent):
        return ui_k(ent, ui_r, ui_c, ui_w)[:N_USR]

    ui0 = ui_agg(entity_emb)

    adj_sum1 = adj_k(user_emb, adj_i, adj_u)
    h_ent, h_ent_res = _finalize(adj_sum1, cnt_adj, entity_emb, N_ENT)
    h_usr, h_usr_res = _user_finalize(ui0, user_emb, hist_int, user_emb)

    edge_sum1 = edge_k(entity_emb, ehead, etail, etype, rel_flat)
    r_ent, r_ent_res = _finalize(edge_sum1, cnt_edge, entity_emb, N_ENT)
    r_usr, r_usr_res = _user_finalize(ui0, user_emb, rel_int, user_emb)

    adj_sum2 = adj_k(h_usr, adj_i, adj_u)
    _, h_ent_res = _finalize(adj_sum2, cnt_adj, h_ent_res, N_ENT)
    _, h_usr_res = _user_finalize(ui_agg(h_ent), h_usr, hist_int, h_usr_res)

    edge_sum2 = edge_k(r_ent, ehead, etail, etype, rel_flat)
    _, r_ent_res = _finalize(edge_sum2, cnt_edge, r_ent_res, N_ENT)
    _, r_usr_res = _user_finalize(ui_agg(r_ent), r_usr, rel_int, r_usr_res)

    entity_res = jnp.concatenate([h_ent_res, r_ent_res], axis=-1)
    user_res = jnp.concatenate([h_usr_res, r_usr_res], axis=-1)
    return (entity_res, user_res, h_ent_res, r_ent_res, h_usr_res, r_usr_res)


# speedup vs baseline: 1.4950x; 1.4950x over previous
import functools

import jax
import jax.numpy as jnp
from jax import lax
from jax.experimental import pallas as pl
from jax.experimental.pallas import tpu as pltpu
from jax.experimental.pallas import tpu_sc as plsc

N_ENT = 100000
N_USR = 50000
N_FAC = 8
D = 128

NC = 2
NS = 16
L = 16

C = 12544
CT = C + 128
KB = 3072
ZR = 16

_CPARAMS = pltpu.CompilerParams(needs_layout_passes=False)
_MESH = dict(core_axis_name="c", subcore_axis_name="s")

_BLK = 1024


def _cdiv(a, b):
    return (a + b - 1) // b


@functools.lru_cache(maxsize=None)
def _build_segsum(n_dst, e_pad, mode):
    n_chunks = _cdiv(n_dst, C)
    out_rows = n_chunks * C
    P = _cdiv(n_chunks, NC)
    Et = e_pad // NS
    nblk = Et // KB
    assert Et % KB == 0
    slice_rows = CT // NS
    wrows = C // NS

    scratch = [
        pltpu.VMEM_SHARED((CT, D), jnp.float32),
        pltpu.VMEM((KB,), jnp.int32),
        pltpu.VMEM((KB,), jnp.int32),
        pltpu.VMEM((KB + 2 * L,), jnp.int32),
        pltpu.VMEM((KB + 2 * L,), jnp.int32),
        pltpu.VMEM((2 * L, D), jnp.float32),
        pltpu.VMEM((ZR, D), jnp.float32),
        pltpu.SemaphoreType.DMA,
    ]
    if mode == "scalar":
        scratch += [pltpu.VMEM((KB,), jnp.float32),
                    pltpu.VMEM((KB + 2 * L,), jnp.float32)]
    elif mode == "rel":
        scratch += [pltpu.VMEM((KB,), jnp.int32),
                    pltpu.VMEM((KB + 2 * L,), jnp.int32),
                    pltpu.VMEM((32 * D,), jnp.float32)]

    def body(*refs):
        if mode == "plain":
            (table, idx_dst_h, idx_src_h, out_h,
             spmem, dst_v, src_v, acc_dst, acc_src, rows, zrow, sem) = refs
            w_v = acc_w = rel_buf = None
        elif mode == "scalar":
            (table, idx_dst_h, idx_src_h, w_h, out_h,
             spmem, dst_v, src_v, acc_dst, acc_src, rows, zrow, sem,
             w_v, acc_w) = refs
            rel_buf = None
        else:
            (table, idx_dst_h, idx_src_h, w_h, rel_h, out_h,
             spmem, dst_v, src_v, acc_dst, acc_src, rows, zrow, sem,
             w_v, acc_w, rel_buf) = refs

        s = lax.axis_index("s")
        c = lax.axis_index("c")
        zero16 = jnp.zeros((L,), jnp.float32)

        for rz in range(ZR):
            for j in range(D // L):
                zrow[rz, pl.ds(j * L, L)] = zero16

        def zinit(i, carry):
            acc_src[pl.ds(i * L, L)] = jnp.zeros((L,), jnp.int32)
            return carry

        lax.fori_loop(0, (KB + 2 * L) // L, zinit, jnp.int32(0))
        if mode == "rel":
            pltpu.sync_copy(rel_h, rel_buf)

        def scale_batch(b, base):
            if mode == "scalar":
                wv = acc_w[pl.ds(b * L, L)]
                for rr in range(L):
                    w = wv[rr]
                    row = base + rr
                    for j in range(D // L):
                        rows[row, pl.ds(j * L, L)] = (
                            rows[row, pl.ds(j * L, L)] * w)
            elif mode == "rel":
                etv = acc_w[pl.ds(b * L, L)]
                for rr in range(L):
                    rb = etv[rr] * D
                    row = base + rr
                    for j in range(D // L):
                        rows[row, pl.ds(j * L, L)] = (
                            rows[row, pl.ds(j * L, L)]
                            * rel_buf[pl.ds(rb + j * L, L)])

        def gfire(b):
            sv = acc_src[pl.ds(b * L, L)]
            base = jnp.bitwise_and(b, 1) * L
            pltpu.async_copy(table.at[sv], rows.at[pl.ds(base, L)], sem)

        def gwait(base):
            pltpu.make_async_copy(table.at[pl.ds(0, L)],
                                  rows.at[pl.ds(base, L)], sem).wait()

        def chunk_body(p, carry):
            chunk = p * NC + c

            @pl.when(chunk < n_chunks)
            def _():
                lo = chunk * C
                hi = lo + C
                zbase = s * slice_rows
                for kk in range(slice_rows // ZR):
                    pltpu.sync_copy(zrow, spmem.at[pl.ds(zbase + kk * ZR, ZR)])
                rem = slice_rows % ZR
                if rem:
                    pltpu.sync_copy(
                        zrow.at[pl.ds(0, rem)],
                        spmem.at[pl.ds(zbase + (slice_rows // ZR) * ZR, rem)])
                plsc.subcore_barrier()

                def block_body(b, carry2):
                    base = s * Et + b * KB
                    pltpu.sync_copy(idx_dst_h.at[pl.ds(base, KB)], dst_v)
                    pltpu.sync_copy(idx_src_h.at[pl.ds(base, KB)], src_v)
                    if mode != "plain":
                        pltpu.sync_copy(w_h.at[pl.ds(base, KB)], w_v)

                    def cvec(k2, n):
                        dv = dst_v[pl.ds(k2 * L, L)]
                        m = (dv >= lo) & (dv < hi)
                        mi = m.astype(jnp.int32)
                        pos = n + plsc.cumsum(mi) - 1
                        plsc.store_scatter(acc_dst, [pos], dv - lo, mask=m)
                        sv = src_v[pl.ds(k2 * L, L)]
                        plsc.store_scatter(acc_src, [pos], sv, mask=m)
                        if mode != "plain":
                            wv = w_v[pl.ds(k2 * L, L)]
                            plsc.store_scatter(acc_w, [pos], wv, mask=m)
                        return n + jnp.sum(mi)

                    n = lax.fori_loop(0, KB // L, cvec, jnp.int32(0))

                    iota = lax.iota(jnp.int32, L)
                    trash = jnp.int32(C) + iota
                    dummy_src = iota * 16 + s * 37
                    acc_dst[pl.ds(n, L)] = trash
                    acc_src[pl.ds(n, L)] = dummy_src
                    if mode != "plain":
                        acc_w[pl.ds(n, L)] = jnp.zeros((L,), acc_w.dtype)

                    nb16 = (n + L - 1) // L

                    @pl.when(nb16 >= 1)
                    def _p0():
                        gfire(jnp.int32(0))

                    @pl.when(nb16 >= 2)
                    def _p1():
                        gfire(jnp.int32(1))

                    def drain(b, carry3):
                        base = jnp.bitwise_and(b, 1) * L
                        gwait(base)
                        scale_batch(b, base)
                        dv = acc_dst[pl.ds(b * L, L)]
                        pltpu.sync_copy(rows.at[pl.ds(base, L)],
                                        spmem.at[dv], add=True)

                        @pl.when(b + 2 < nb16)
                        def _():
                            gfire(b + 2)

                        return carry3

                    lax.fori_loop(0, nb16, drain, jnp.int32(0))
                    return carry2

                lax.fori_loop(0, nblk, block_body, jnp.int32(0))
                plsc.subcore_barrier()
                pltpu.sync_copy(
                    spmem.at[pl.ds(s * wrows, wrows)],
                    out_h.at[pl.ds(chunk * C + s * wrows, wrows)])
                plsc.subcore_barrier()

            return carry

        lax.fori_loop(0, P, chunk_body, jnp.int32(0))

    return pl.kernel(
        body,
        out_type=jax.ShapeDtypeStruct((out_rows, D), jnp.float32),
        mesh=plsc.VectorSubcoreMesh(**_MESH),
        scratch_types=scratch,
        compiler_params=_CPARAMS,
    )


@functools.lru_cache(maxsize=None)
def _build_counts(n_dst_pad, e1_pad, e2_pad):
    buf = n_dst_pad + 8 * L
    assert buf % (NS * 8) == 0
    per_tile = buf // NS
    wr = n_dst_pad // NS

    def body(idx1_h, idx2_h, out_h, cntA, cntB, idx_v, ones_v, zv):
        s = lax.axis_index("s")
        c = lax.axis_index("c")

        def fill(i, carry):
            zv[pl.ds(i * L, L)] = jnp.zeros((L,), jnp.float32)
            ones_v[pl.ds(i * L, L)] = jnp.ones((L,), jnp.float32)
            return carry

        lax.fori_loop(0, KB // L, fill, jnp.int32(0))

        zbase = s * per_tile
        nz = per_tile // KB
        for cnt in (cntA, cntB):
            for kk in range(nz):
                pltpu.sync_copy(zv, cnt.at[pl.ds(zbase + kk * KB, KB)])
            rem = per_tile % KB
            if rem:
                pltpu.sync_copy(zv.at[pl.ds(0, rem)],
                                cnt.at[pl.ds(zbase + nz * KB, rem)])
        plsc.subcore_barrier()

        for idx_h, cnt, e_pad in ((idx1_h, cntA, e1_pad),
                                  (idx2_h, cntB, e2_pad)):
            Et = e_pad // (NC * NS)
            nblk = Et // KB
            assert Et % KB == 0

            def block_body(b, carry, idx_h=idx_h, cnt=cnt, Et=Et):
                base = (c * NS + s) * Et + b * KB
                pltpu.sync_copy(idx_h.at[pl.ds(base, KB)], idx_v)
                pltpu.sync_copy(ones_v, cnt.at[idx_v], add=True)
                return carry

            lax.fori_loop(0, nblk, block_body, jnp.int32(0))
        plsc.subcore_barrier()
        pltpu.sync_copy(cntA.at[pl.ds(s * wr, wr)],
                        out_h.at[c, pl.ds(s * wr, wr)])
        pltpu.sync_copy(cntB.at[pl.ds(s * wr, wr)],
                        out_h.at[c, pl.ds(n_dst_pad + s * wr, wr)])

    return pl.kernel(
        body,
        out_type=jax.ShapeDtypeStruct((NC, 2 * n_dst_pad), jnp.float32),
        mesh=plsc.VectorSubcoreMesh(**_MESH),
        scratch_types=[
            pltpu.VMEM_SHARED((buf,), jnp.float32),
            pltpu.VMEM_SHARED((buf,), jnp.float32),
            pltpu.VMEM((KB,), jnp.int32),
            pltpu.VMEM((KB,), jnp.float32),
            pltpu.VMEM((KB,), jnp.float32),
        ],
        compiler_params=_CPARAMS,
    )


def _finalize_body(sum_ref, cnt_ref, res_ref, emb_out_ref, res_out_ref):
    sval = sum_ref[...]
    cval = cnt_ref[...]
    mean = sval / jnp.maximum(cval, 1.0)
    nrm = jnp.sqrt(jnp.sum(mean * mean, axis=-1, keepdims=True))
    emb = mean / jnp.maximum(nrm, 1e-12)
    emb_out_ref[...] = emb
    res_out_ref[...] = res_ref[...] + emb


def _finalize(seg_sum, cnt, res, n_rows):
    pad = (-n_rows) % _BLK
    seg_sum = seg_sum[:n_rows]
    if pad:
        seg_sum = jnp.pad(seg_sum, ((0, pad), (0, 0)))
        cnt = jnp.pad(cnt, ((0, pad), (0, 0)))
        res = jnp.pad(res, ((0, pad), (0, 0)))
    n_pad = n_rows + pad
    emb, res_out = pl.pallas_call(
        _finalize_body,
        grid=(n_pad // _BLK,),
        in_specs=[
            pl.BlockSpec((_BLK, D), lambda i: (i, 0)),
            pl.BlockSpec((_BLK, 1), lambda i: (i, 0)),
            pl.BlockSpec((_BLK, D), lambda i: (i, 0)),
        ],
        out_specs=[
            pl.BlockSpec((_BLK, D), lambda i: (i, 0)),
            pl.BlockSpec((_BLK, D), lambda i: (i, 0)),
        ],
        out_shape=[
            jax.ShapeDtypeStruct((n_pad, D), jnp.float32),
            jax.ShapeDtypeStruct((n_pad, D), jnp.float32),
        ],
    )(seg_sum, cnt, res)
    return emb[:n_rows], res_out[:n_rows]


def _user_finalize(user_agg, usr, intent, res):
    score = jax.nn.softmax(usr @ intent.T, axis=1)
    usr_agg = (score @ intent) * user_agg + user_agg
    ones = jnp.ones((usr_agg.shape[0], 1), jnp.float32)
    return _finalize(usr_agg, ones, res, usr_agg.shape[0])


def _pad_to(x, e_pad, fill):
    if x.shape[0] == e_pad:
        return x
    return jnp.concatenate([x, fill])


def _pad_edges(dst, src, w, n_dst_pad, n_src):
    e = dst.shape[0]
    step = NC * NS * KB
    e_pad = _cdiv(e, step) * step
    npad = e_pad - e
    ar = lax.iota(jnp.int32, npad)
    dstp = _pad_to(dst.astype(jnp.int32), e_pad,
                   n_dst_pad + (ar % (8 * L)))
    srcp = _pad_to(src.astype(jnp.int32), e_pad, ar % n_src)
    wp = None
    if w is not None:
        fill = (jnp.zeros((npad,), w.dtype) if w.dtype == jnp.float32
                else (ar % 32).astype(w.dtype))
        wp = _pad_to(w, e_pad, fill)
    return dstp, srcp, wp, e_pad


def kernel(user_emb, entity_emb, rel_weight, rel_intent_proj, hist_intent,
           ui_vals, edge_index, edge_type, ui_rows, ui_cols,
           adj_user_idx, adj_item_idx):
    relation_emb = rel_weight
    rel_int = rel_intent_proj @ relation_emb
    hist_int = hist_intent
    head, tail = edge_index[0], edge_index[1]

    n_ent_chunks = _cdiv(N_ENT, C)
    n_usr_chunks = _cdiv(N_USR, C)
    ent_pad = n_ent_chunks * C
    usr_pad = n_usr_chunks * C

    ui_r, ui_c, ui_w, e_ui = _pad_edges(ui_rows, ui_cols, ui_vals,
                                        usr_pad, N_ENT)
    adj_i, adj_u, _, e_adj = _pad_edges(adj_item_idx, adj_user_idx, None,
                                        ent_pad, N_USR)
    ehead, etail, etype, e_edge = _pad_edges(head, tail, edge_type,
                                             ent_pad, N_ENT)
    rel_flat = relation_emb.reshape((32 * D,))

    cnt2 = _build_counts(ent_pad, e_adj, e_edge)(adj_i, ehead)
    cnts = cnt2[0] + cnt2[1]
    cnt_adj = cnts[:N_ENT, None]
    cnt_edge = cnts[ent_pad:ent_pad + N_ENT, None]

    ui_k = _build_segsum(N_USR, e_ui, "scalar")
    adj_k = _build_segsum(N_ENT, e_adj, "plain")
    edge_k = _build_segsum(N_ENT, e_edge, "rel")

    def ui_agg(ent):
        return ui_k(ent, ui_r, ui_c, ui_w)[:N_USR]

    ui0 = ui_agg(entity_emb)

    adj_sum1 = adj_k(user_emb, adj_i, adj_u)
    h_ent, h_ent_res = _finalize(adj_sum1, cnt_adj, entity_emb, N_ENT)
    h_usr, h_usr_res = _user_finalize(ui0, user_emb, hist_int, user_emb)

    edge_sum1 = edge_k(entity_emb, ehead, etail, etype, rel_flat)
    r_ent, r_ent_res = _finalize(edge_sum1, cnt_edge, entity_emb, N_ENT)
    r_usr, r_usr_res = _user_finalize(ui0, user_emb, rel_int, user_emb)

    adj_sum2 = adj_k(h_usr, adj_i, adj_u)
    _, h_ent_res = _finalize(adj_sum2, cnt_adj, h_ent_res, N_ENT)
    _, h_usr_res = _user_finalize(ui_agg(h_ent), h_usr, hist_int, h_usr_res)

    edge_sum2 = edge_k(r_ent, ehead, etail, etype, rel_flat)
    _, r_ent_res = _finalize(edge_sum2, cnt_edge, r_ent_res, N_ENT)
    _, r_usr_res = _user_finalize(ui_agg(r_ent), r_usr, rel_int, r_usr_res)

    entity_res = jnp.concatenate([h_ent_res, r_ent_res], axis=-1)
    user_res = jnp.concatenate([h_usr_res, r_usr_res], axis=-1)
    return (entity_res, user_res, h_ent_res, r_ent_res, h_usr_res, r_usr_res)


# speedup vs baseline: 1.6288x; 1.0895x over previous
import functools

import jax
import jax.numpy as jnp
from jax import lax
from jax.experimental import pallas as pl
from jax.experimental.pallas import tpu as pltpu
from jax.experimental.pallas import tpu_sc as plsc

N_ENT = 100000
N_USR = 50000
N_FAC = 8
D = 128

NC = 2
NS = 16
L = 16

C = 12544
CT = C + 128
KB = 2560
ZR = 8

_CPARAMS = pltpu.CompilerParams(needs_layout_passes=False)
_MESH = dict(core_axis_name="c", subcore_axis_name="s")

_BLK = 1024


def _cdiv(a, b):
    return (a + b - 1) // b


@functools.lru_cache(maxsize=None)
def _build_segsum(n_dst, e_pad, mode):
    n_chunks = _cdiv(n_dst, C)
    out_rows = n_chunks * C
    P = _cdiv(n_chunks, NC)
    Et = e_pad // NS
    nblk = Et // KB
    assert Et % KB == 0
    slice_rows = CT // NS
    wrows = C // NS

    scratch = [
        pltpu.VMEM_SHARED((CT, D), jnp.float32),
        pltpu.VMEM((KB,), jnp.int32),
        pltpu.VMEM((KB,), jnp.int32),
        pltpu.VMEM((KB + 2 * L,), jnp.int32),
        pltpu.VMEM((KB + 2 * L,), jnp.int32),
        pltpu.VMEM((4 * L, D), jnp.float32),
        pltpu.VMEM((ZR, D), jnp.float32),
        pltpu.SemaphoreType.DMA,
        pltpu.SemaphoreType.DMA,
    ]
    if mode == "scalar":
        scratch += [pltpu.VMEM((KB,), jnp.float32),
                    pltpu.VMEM((KB + 2 * L,), jnp.float32)]
    elif mode == "rel":
        scratch += [pltpu.VMEM((KB,), jnp.int32),
                    pltpu.VMEM((KB + 2 * L,), jnp.int32),
                    pltpu.VMEM((32 * D,), jnp.float32)]

    def body(*refs):
        if mode == "plain":
            (table, idx_dst_h, idx_src_h, out_h,
             spmem, dst_v, src_v, acc_dst, acc_src, rows, zrow,
             sem, sem2) = refs
            w_v = acc_w = rel_buf = None
        elif mode == "scalar":
            (table, idx_dst_h, idx_src_h, w_h, out_h,
             spmem, dst_v, src_v, acc_dst, acc_src, rows, zrow, sem, sem2,
             w_v, acc_w) = refs
            rel_buf = None
        else:
            (table, idx_dst_h, idx_src_h, w_h, rel_h, out_h,
             spmem, dst_v, src_v, acc_dst, acc_src, rows, zrow, sem, sem2,
             w_v, acc_w, rel_buf) = refs

        s = lax.axis_index("s")
        c = lax.axis_index("c")
        zero16 = jnp.zeros((L,), jnp.float32)

        for rz in range(ZR):
            for j in range(D // L):
                zrow[rz, pl.ds(j * L, L)] = zero16

        def zinit(i, carry):
            acc_src[pl.ds(i * L, L)] = jnp.zeros((L,), jnp.int32)
            return carry

        lax.fori_loop(0, (KB + 2 * L) // L, zinit, jnp.int32(0))
        if mode == "rel":
            pltpu.sync_copy(rel_h, rel_buf)

        def scale_batch(b, base):
            if mode == "scalar":
                wv = acc_w[pl.ds(b * L, L)]
                for rr in range(L):
                    w = wv[rr]
                    row = base + rr
                    for j in range(D // L):
                        rows[row, pl.ds(j * L, L)] = (
                            rows[row, pl.ds(j * L, L)] * w)
            elif mode == "rel":
                etv = acc_w[pl.ds(b * L, L)]
                for rr in range(L):
                    rb = etv[rr] * D
                    row = base + rr
                    for j in range(D // L):
                        rows[row, pl.ds(j * L, L)] = (
                            rows[row, pl.ds(j * L, L)]
                            * rel_buf[pl.ds(rb + j * L, L)])

        def gfire(b):
            sv = acc_src[pl.ds(b * L, L)]
            base = jnp.bitwise_and(b, 3) * L
            pltpu.async_copy(table.at[sv], rows.at[pl.ds(base, L)], sem)

        def gwait(base):
            pltpu.make_async_copy(table.at[pl.ds(0, L)],
                                  rows.at[pl.ds(base, L)], sem).wait()

        def swait():
            pltpu.make_async_copy(rows.at[pl.ds(0, L)],
                                  spmem.at[pl.ds(0, L)], sem2).wait()

        def chunk_body(p, carry):
            chunk = p * NC + c

            @pl.when(chunk < n_chunks)
            def _():
                lo = chunk * C
                hi = lo + C
                zbase = s * slice_rows
                for kk in range(slice_rows // ZR):
                    pltpu.sync_copy(zrow, spmem.at[pl.ds(zbase + kk * ZR, ZR)])
                rem = slice_rows % ZR
                if rem:
                    pltpu.sync_copy(
                        zrow.at[pl.ds(0, rem)],
                        spmem.at[pl.ds(zbase + (slice_rows // ZR) * ZR, rem)])
                plsc.subcore_barrier()

                def block_body(b, carry2):
                    base = s * Et + b * KB
                    pltpu.sync_copy(idx_dst_h.at[pl.ds(base, KB)], dst_v)
                    pltpu.sync_copy(idx_src_h.at[pl.ds(base, KB)], src_v)
                    if mode != "plain":
                        pltpu.sync_copy(w_h.at[pl.ds(base, KB)], w_v)

                    def cvec(k2, n):
                        dv = dst_v[pl.ds(k2 * L, L)]
                        m = (dv >= lo) & (dv < hi)
                        mi = m.astype(jnp.int32)
                        pos = n + plsc.cumsum(mi) - 1
                        plsc.store_scatter(acc_dst, [pos], dv - lo, mask=m)
                        sv = src_v[pl.ds(k2 * L, L)]
                        plsc.store_scatter(acc_src, [pos], sv, mask=m)
                        if mode != "plain":
                            wv = w_v[pl.ds(k2 * L, L)]
                            plsc.store_scatter(acc_w, [pos], wv, mask=m)
                        return n + jnp.sum(mi)

                    n = lax.fori_loop(0, KB // L, cvec, jnp.int32(0))

                    iota = lax.iota(jnp.int32, L)
                    trash = jnp.int32(C) + iota
                    dummy_src = iota * 16 + s * 37
                    acc_dst[pl.ds(n, L)] = trash
                    acc_src[pl.ds(n, L)] = dummy_src
                    if mode != "plain":
                        acc_w[pl.ds(n, L)] = jnp.zeros((L,), acc_w.dtype)

                    nb16 = (n + L - 1) // L

                    @pl.when(nb16 >= 1)
                    def _p0():
                        gfire(jnp.int32(0))

                    @pl.when(nb16 >= 2)
                    def _p1():
                        gfire(jnp.int32(1))

                    def drain(b, carry3):
                        base = jnp.bitwise_and(b, 3) * L
                        gwait(base)
                        scale_batch(b, base)
                        dv = acc_dst[pl.ds(b * L, L)]
                        pltpu.async_copy(rows.at[pl.ds(base, L)],
                                         spmem.at[dv], sem2, add=True)

                        @pl.when(b >= 2)
                        def _():
                            swait()

                        @pl.when(b + 2 < nb16)
                        def _():
                            gfire(b + 2)

                        return carry3

                    lax.fori_loop(0, nb16, drain, jnp.int32(0))

                    @pl.when(nb16 >= 1)
                    def _t0():
                        swait()

                    @pl.when(nb16 >= 2)
                    def _t1():
                        swait()

                    return carry2

                lax.fori_loop(0, nblk, block_body, jnp.int32(0))
                plsc.subcore_barrier()
                pltpu.sync_copy(
                    spmem.at[pl.ds(s * wrows, wrows)],
                    out_h.at[pl.ds(chunk * C + s * wrows, wrows)])
                plsc.subcore_barrier()

            return carry

        lax.fori_loop(0, P, chunk_body, jnp.int32(0))

    return pl.kernel(
        body,
        out_type=jax.ShapeDtypeStruct((out_rows, D), jnp.float32),
        mesh=plsc.VectorSubcoreMesh(**_MESH),
        scratch_types=scratch,
        compiler_params=_CPARAMS,
    )


@functools.lru_cache(maxsize=None)
def _build_counts(n_dst_pad, e1_pad, e2_pad):
    buf = n_dst_pad + 8 * L
    assert buf % (NS * 8) == 0
    per_tile = buf // NS
    wr = n_dst_pad // NS

    def body(idx1_h, idx2_h, out_h, cntA, cntB, idx_v, ones_v, zv):
        s = lax.axis_index("s")
        c = lax.axis_index("c")

        def fill(i, carry):
            zv[pl.ds(i * L, L)] = jnp.zeros((L,), jnp.float32)
            ones_v[pl.ds(i * L, L)] = jnp.ones((L,), jnp.float32)
            return carry

        lax.fori_loop(0, KB // L, fill, jnp.int32(0))

        zbase = s * per_tile
        nz = per_tile // KB
        for cnt in (cntA, cntB):
            for kk in range(nz):
                pltpu.sync_copy(zv, cnt.at[pl.ds(zbase + kk * KB, KB)])
            rem = per_tile % KB
            if rem:
                pltpu.sync_copy(zv.at[pl.ds(0, rem)],
                                cnt.at[pl.ds(zbase + nz * KB, rem)])
        plsc.subcore_barrier()

        for idx_h, cnt, e_pad in ((idx1_h, cntA, e1_pad),
                                  (idx2_h, cntB, e2_pad)):
            Et = e_pad // (NC * NS)
            nblk = Et // KB
            assert Et % KB == 0

            def block_body(b, carry, idx_h=idx_h, cnt=cnt, Et=Et):
                base = (c * NS + s) * Et + b * KB
                pltpu.sync_copy(idx_h.at[pl.ds(base, KB)], idx_v)
                pltpu.sync_copy(ones_v, cnt.at[idx_v], add=True)
                return carry

            lax.fori_loop(0, nblk, block_body, jnp.int32(0))
        plsc.subcore_barrier()
        pltpu.sync_copy(cntA.at[pl.ds(s * wr, wr)],
                        out_h.at[c, pl.ds(s * wr, wr)])
        pltpu.sync_copy(cntB.at[pl.ds(s * wr, wr)],
                        out_h.at[c, pl.ds(n_dst_pad + s * wr, wr)])

    return pl.kernel(
        body,
        out_type=jax.ShapeDtypeStruct((NC, 2 * n_dst_pad), jnp.float32),
        mesh=plsc.VectorSubcoreMesh(**_MESH),
        scratch_types=[
            pltpu.VMEM_SHARED((buf,), jnp.float32),
            pltpu.VMEM_SHARED((buf,), jnp.float32),
            pltpu.VMEM((KB,), jnp.int32),
            pltpu.VMEM((KB,), jnp.float32),
            pltpu.VMEM((KB,), jnp.float32),
        ],
        compiler_params=_CPARAMS,
    )


def _finalize_body(sum_ref, cnt_ref, res_ref, emb_out_ref, res_out_ref):
    sval = sum_ref[...]
    cval = cnt_ref[...]
    mean = sval / jnp.maximum(cval, 1.0)
    nrm = jnp.sqrt(jnp.sum(mean * mean, axis=-1, keepdims=True))
    emb = mean / jnp.maximum(nrm, 1e-12)
    emb_out_ref[...] = emb
    res_out_ref[...] = res_ref[...] + emb


def _finalize(seg_sum, cnt, res, n_rows):
    pad = (-n_rows) % _BLK
    seg_sum = seg_sum[:n_rows]
    if pad:
        seg_sum = jnp.pad(seg_sum, ((0, pad), (0, 0)))
        cnt = jnp.pad(cnt, ((0, pad), (0, 0)))
        res = jnp.pad(res, ((0, pad), (0, 0)))
    n_pad = n_rows + pad
    emb, res_out = pl.pallas_call(
        _finalize_body,
        grid=(n_pad // _BLK,),
        in_specs=[
            pl.BlockSpec((_BLK, D), lambda i: (i, 0)),
            pl.BlockSpec((_BLK, 1), lambda i: (i, 0)),
            pl.BlockSpec((_BLK, D), lambda i: (i, 0)),
        ],
        out_specs=[
            pl.BlockSpec((_BLK, D), lambda i: (i, 0)),
            pl.BlockSpec((_BLK, D), lambda i: (i, 0)),
        ],
        out_shape=[
            jax.ShapeDtypeStruct((n_pad, D), jnp.float32),
            jax.ShapeDtypeStruct((n_pad, D), jnp.float32),
        ],
    )(seg_sum, cnt, res)
    return emb[:n_rows], res_out[:n_rows]


def _user_finalize(user_agg, usr, intent, res):
    score = jax.nn.softmax(usr @ intent.T, axis=1)
    usr_agg = (score @ intent) * user_agg + user_agg
    ones = jnp.ones((usr_agg.shape[0], 1), jnp.float32)
    return _finalize(usr_agg, ones, res, usr_agg.shape[0])


def _pad_to(x, e_pad, fill):
    if x.shape[0] == e_pad:
        return x
    return jnp.concatenate([x, fill])


def _pad_edges(dst, src, w, n_dst_pad, n_src):
    e = dst.shape[0]
    step = NC * NS * KB
    e_pad = _cdiv(e, step) * step
    npad = e_pad - e
    ar = lax.iota(jnp.int32, npad)
    dstp = _pad_to(dst.astype(jnp.int32), e_pad,
                   n_dst_pad + (ar % (8 * L)))
    srcp = _pad_to(src.astype(jnp.int32), e_pad, ar % n_src)
    wp = None
    if w is not None:
        fill = (jnp.zeros((npad,), w.dtype) if w.dtype == jnp.float32
                else (ar % 32).astype(w.dtype))
        wp = _pad_to(w, e_pad, fill)
    return dstp, srcp, wp, e_pad


def kernel(user_emb, entity_emb, rel_weight, rel_intent_proj, hist_intent,
           ui_vals, edge_index, edge_type, ui_rows, ui_cols,
           adj_user_idx, adj_item_idx):
    relation_emb = rel_weight
    rel_int = rel_intent_proj @ relation_emb
    hist_int = hist_intent
    head, tail = edge_index[0], edge_index[1]

    n_ent_chunks = _cdiv(N_ENT, C)
    n_usr_chunks = _cdiv(N_USR, C)
    ent_pad = n_ent_chunks * C
    usr_pad = n_usr_chunks * C

    ui_r, ui_c, ui_w, e_ui = _pad_edges(ui_rows, ui_cols, ui_vals,
                                        usr_pad, N_ENT)
    adj_i, adj_u, _, e_adj = _pad_edges(adj_item_idx, adj_user_idx, None,
                                        ent_pad, N_USR)
    ehead, etail, etype, e_edge = _pad_edges(head, tail, edge_type,
                                             ent_pad, N_ENT)
    rel_flat = relation_emb.reshape((32 * D,))

    cnt2 = _build_counts(ent_pad, e_adj, e_edge)(adj_i, ehead)
    cnts = cnt2[0] + cnt2[1]
    cnt_adj = cnts[:N_ENT, None]
    cnt_edge = cnts[ent_pad:ent_pad + N_ENT, None]

    ui_k = _build_segsum(N_USR, e_ui, "scalar")
    adj_k = _build_segsum(N_ENT, e_adj, "plain")
    edge_k = _build_segsum(N_ENT, e_edge, "rel")

    def ui_agg(ent):
        return ui_k(ent, ui_r, ui_c, ui_w)[:N_USR]

    ui0 = ui_agg(entity_emb)

    adj_sum1 = adj_k(user_emb, adj_i, adj_u)
    h_ent, h_ent_res = _finalize(adj_sum1, cnt_adj, entity_emb, N_ENT)
    h_usr, h_usr_res = _user_finalize(ui0, user_emb, hist_int, user_emb)

    edge_sum1 = edge_k(entity_emb, ehead, etail, etype, rel_flat)
    r_ent, r_ent_res = _finalize(edge_sum1, cnt_edge, entity_emb, N_ENT)
    r_usr, r_usr_res = _user_finalize(ui0, user_emb, rel_int, user_emb)

    adj_sum2 = adj_k(h_usr, adj_i, adj_u)
    _, h_ent_res = _finalize(adj_sum2, cnt_adj, h_ent_res, N_ENT)
    _, h_usr_res = _user_finalize(ui_agg(h_ent), h_usr, hist_int, h_usr_res)

    edge_sum2 = edge_k(r_ent, ehead, etail, etype, rel_flat)
    _, r_ent_res = _finalize(edge_sum2, cnt_edge, r_ent_res, N_ENT)
    _, r_usr_res = _user_finalize(ui_agg(r_ent), r_usr, rel_int, r_usr_res)

    entity_res = jnp.concatenate([h_ent_res, r_ent_res], axis=-1)
    user_res = jnp.concatenate([h_usr_res, r_usr_res], axis=-1)
    return (entity_res, user_res, h_ent_res, r_ent_res, h_usr_res, r_usr_res)


# speedup vs baseline: 1.8112x; 1.1120x over previous
import functools

import jax
import jax.numpy as jnp
from jax import lax
from jax.experimental import pallas as pl
from jax.experimental.pallas import tpu as pltpu
from jax.experimental.pallas import tpu_sc as plsc

N_ENT = 100000
N_USR = 50000
N_FAC = 8
D = 128

NC = 2
NS = 16
L = 16

C = 12544
CT = C + 128
KB = 2048
ZR = 4
B = 32
NSLOT = 3

_CPARAMS = pltpu.CompilerParams(needs_layout_passes=False)
_MESH = dict(core_axis_name="c", subcore_axis_name="s")

_BLK = 1024


def _cdiv(a, b):
    return (a + b - 1) // b


@functools.lru_cache(maxsize=None)
def _build_segsum(n_dst, e_pad, mode):
    n_chunks = _cdiv(n_dst, C)
    out_rows = n_chunks * C
    P = _cdiv(n_chunks, NC)
    Et = e_pad // NS
    nblk = Et // KB
    assert Et % KB == 0
    slice_rows = CT // NS
    wrows = C // NS

    scratch = [
        pltpu.VMEM_SHARED((CT, D), jnp.float32),
        pltpu.VMEM((KB,), jnp.int32),
        pltpu.VMEM((KB,), jnp.int32),
        pltpu.VMEM((KB + 2 * L,), jnp.int32),
        pltpu.VMEM((KB + 2 * L,), jnp.int32),
        pltpu.VMEM((NSLOT * B, D), jnp.float32),
        pltpu.VMEM((ZR, D), jnp.float32),
        pltpu.SemaphoreType.DMA,
        pltpu.SemaphoreType.DMA,
    ]
    if mode == "scalar":
        scratch += [pltpu.VMEM((KB,), jnp.float32),
                    pltpu.VMEM((KB + 2 * L,), jnp.float32)]
    elif mode == "rel":
        scratch += [pltpu.VMEM((KB,), jnp.int32),
                    pltpu.VMEM((KB + 2 * L,), jnp.int32),
                    pltpu.VMEM((32 * D,), jnp.float32)]

    def body(*refs):
        if mode == "plain":
            (table, idx_dst_h, idx_src_h, out_h,
             spmem, dst_v, src_v, acc_dst, acc_src, rows, zrow,
             sem, sem2) = refs
            w_v = acc_w = rel_buf = None
        elif mode == "scalar":
            (table, idx_dst_h, idx_src_h, w_h, out_h,
             spmem, dst_v, src_v, acc_dst, acc_src, rows, zrow, sem, sem2,
             w_v, acc_w) = refs
            rel_buf = None
        else:
            (table, idx_dst_h, idx_src_h, w_h, rel_h, out_h,
             spmem, dst_v, src_v, acc_dst, acc_src, rows, zrow, sem, sem2,
             w_v, acc_w, rel_buf) = refs

        s = lax.axis_index("s")
        c = lax.axis_index("c")
        zero16 = jnp.zeros((L,), jnp.float32)

        for rz in range(ZR):
            for j in range(D // L):
                zrow[rz, pl.ds(j * L, L)] = zero16

        def zinit(i, carry):
            acc_src[pl.ds(i * L, L)] = jnp.zeros((L,), jnp.int32)
            return carry

        lax.fori_loop(0, (KB + 2 * L) // L, zinit, jnp.int32(0))
        if mode == "rel":
            pltpu.sync_copy(rel_h, rel_buf)

        def scale_batch(k, base):
            if mode == "plain":
                return
            for half in range(B // L):
                wv = acc_w[pl.ds(k * B + half * L, L)]
                for rr in range(L):
                    row = base + half * L + rr
                    if mode == "scalar":
                        w = wv[rr]
                        for j in range(D // L):
                            rows[row, pl.ds(j * L, L)] = (
                                rows[row, pl.ds(j * L, L)] * w)
                    else:
                        rb = wv[rr] * D
                        for j in range(D // L):
                            rows[row, pl.ds(j * L, L)] = (
                                rows[row, pl.ds(j * L, L)]
                                * rel_buf[pl.ds(rb + j * L, L)])

        def gfire(k):
            slot = lax.rem(k, NSLOT) * B
            pltpu.async_copy(table.at[acc_src.at[pl.ds(k * B, B)]],
                             rows.at[pl.ds(slot, B)], sem)

        def gwait(slot):
            pltpu.make_async_copy(table.at[pl.ds(0, B)],
                                  rows.at[pl.ds(slot, B)], sem).wait()

        def swait():
            pltpu.make_async_copy(rows.at[pl.ds(0, B)],
                                  spmem.at[pl.ds(0, B)], sem2).wait()

        def chunk_body(p, carry):
            chunk = p * NC + c

            @pl.when(chunk < n_chunks)
            def _():
                lo = chunk * C
                hi = lo + C
                zbase = s * slice_rows
                for kk in range(slice_rows // ZR):
                    pltpu.sync_copy(zrow, spmem.at[pl.ds(zbase + kk * ZR, ZR)])
                rem = slice_rows % ZR
                if rem:
                    pltpu.sync_copy(
                        zrow.at[pl.ds(0, rem)],
                        spmem.at[pl.ds(zbase + (slice_rows // ZR) * ZR, rem)])
                plsc.subcore_barrier()

                def block_body(b, carry2):
                    base = s * Et + b * KB
                    pltpu.sync_copy(idx_dst_h.at[pl.ds(base, KB)], dst_v)
                    pltpu.sync_copy(idx_src_h.at[pl.ds(base, KB)], src_v)
                    if mode != "plain":
                        pltpu.sync_copy(w_h.at[pl.ds(base, KB)], w_v)

                    def cvec(k2, n):
                        dv = dst_v[pl.ds(k2 * L, L)]
                        m = (dv >= lo) & (dv < hi)
                        mi = m.astype(jnp.int32)
                        pos = n + plsc.cumsum(mi) - 1
                        plsc.store_scatter(acc_dst, [pos], dv - lo, mask=m)
                        sv = src_v[pl.ds(k2 * L, L)]
                        plsc.store_scatter(acc_src, [pos], sv, mask=m)
                        if mode != "plain":
                            wv = w_v[pl.ds(k2 * L, L)]
                            plsc.store_scatter(acc_w, [pos], wv, mask=m)
                        return n + jnp.sum(mi)

                    n = lax.fori_loop(0, KB // L, cvec, jnp.int32(0))

                    iota = lax.iota(jnp.int32, L)
                    dummy_src = iota * 16 + s * 37
                    for t in range(B // L):
                        acc_dst[pl.ds(n + t * L, L)] = (
                            jnp.int32(C + t * L) + iota)
                        acc_src[pl.ds(n + t * L, L)] = dummy_src
                        if mode != "plain":
                            acc_w[pl.ds(n + t * L, L)] = jnp.zeros(
                                (L,), acc_w.dtype)

                    nb = (n + B - 1) // B

                    @pl.when(nb >= 1)
                    def _p0():
                        gfire(jnp.int32(0))

                    @pl.when(nb >= 2)
                    def _p1():
                        gfire(jnp.int32(1))

                    def drain(k, carry3):
                        slot = lax.rem(k, NSLOT) * B
                        gwait(slot)
                        scale_batch(k, slot)
                        pltpu.async_copy(
                            rows.at[pl.ds(slot, B)],
                            spmem.at[acc_dst.at[pl.ds(k * B, B)]],
                            sem2, add=True)

                        @pl.when(k >= 1)
                        def _():
                            swait()

                        @pl.when(k + 2 < nb)
                        def _():
                            gfire(k + 2)

                        return carry3

                    lax.fori_loop(0, nb, drain, jnp.int32(0))

                    @pl.when(nb >= 1)
                    def _t0():
                        swait()

                    return carry2

                lax.fori_loop(0, nblk, block_body, jnp.int32(0))
                plsc.subcore_barrier()
                pltpu.sync_copy(
                    spmem.at[pl.ds(s * wrows, wrows)],
                    out_h.at[pl.ds(chunk * C + s * wrows, wrows)])
                plsc.subcore_barrier()

            return carry

        lax.fori_loop(0, P, chunk_body, jnp.int32(0))

    return pl.kernel(
        body,
        out_type=jax.ShapeDtypeStruct((out_rows, D), jnp.float32),
        mesh=plsc.VectorSubcoreMesh(**_MESH),
        scratch_types=scratch,
        compiler_params=_CPARAMS,
    )


@functools.lru_cache(maxsize=None)
def _build_counts(n_dst_pad, e1_pad, e2_pad):
    buf = n_dst_pad + 8 * L
    assert buf % (NS * 8) == 0
    per_tile = buf // NS
    wr = n_dst_pad // NS

    def body(idx1_h, idx2_h, out_h, cntA, cntB, idx_v, ones_v, zv):
        s = lax.axis_index("s")
        c = lax.axis_index("c")

        def fill(i, carry):
            zv[pl.ds(i * L, L)] = jnp.zeros((L,), jnp.float32)
            ones_v[pl.ds(i * L, L)] = jnp.ones((L,), jnp.float32)
            return carry

        lax.fori_loop(0, KB // L, fill, jnp.int32(0))

        zbase = s * per_tile
        nz = per_tile // KB
        for cnt in (cntA, cntB):
            for kk in range(nz):
                pltpu.sync_copy(zv, cnt.at[pl.ds(zbase + kk * KB, KB)])
            rem = per_tile % KB
            if rem:
                pltpu.sync_copy(zv.at[pl.ds(0, rem)],
                                cnt.at[pl.ds(zbase + nz * KB, rem)])
        plsc.subcore_barrier()

        for idx_h, cnt, e_pad in ((idx1_h, cntA, e1_pad),
                                  (idx2_h, cntB, e2_pad)):
            Et = e_pad // (NC * NS)
            nblk = Et // KB
            assert Et % KB == 0

            def block_body(b, carry, idx_h=idx_h, cnt=cnt, Et=Et):
                base = (c * NS + s) * Et + b * KB
                pltpu.sync_copy(idx_h.at[pl.ds(base, KB)], idx_v)
                pltpu.sync_copy(ones_v, cnt.at[idx_v], add=True)
                return carry

            lax.fori_loop(0, nblk, block_body, jnp.int32(0))
        plsc.subcore_barrier()
        pltpu.sync_copy(cntA.at[pl.ds(s * wr, wr)],
                        out_h.at[c, pl.ds(s * wr, wr)])
        pltpu.sync_copy(cntB.at[pl.ds(s * wr, wr)],
                        out_h.at[c, pl.ds(n_dst_pad + s * wr, wr)])

    return pl.kernel(
        body,
        out_type=jax.ShapeDtypeStruct((NC, 2 * n_dst_pad), jnp.float32),
        mesh=plsc.VectorSubcoreMesh(**_MESH),
        scratch_types=[
            pltpu.VMEM_SHARED((buf,), jnp.float32),
            pltpu.VMEM_SHARED((buf,), jnp.float32),
            pltpu.VMEM((KB,), jnp.int32),
            pltpu.VMEM((KB,), jnp.float32),
            pltpu.VMEM((KB,), jnp.float32),
        ],
        compiler_params=_CPARAMS,
    )


def _finalize_body(sum_ref, cnt_ref, res_ref, emb_out_ref, res_out_ref):
    sval = sum_ref[...]
    cval = cnt_ref[...]
    mean = sval / jnp.maximum(cval, 1.0)
    nrm = jnp.sqrt(jnp.sum(mean * mean, axis=-1, keepdims=True))
    emb = mean / jnp.maximum(nrm, 1e-12)
    emb_out_ref[...] = emb
    res_out_ref[...] = res_ref[...] + emb


def _finalize(seg_sum, cnt, res, n_rows):
    pad = (-n_rows) % _BLK
    seg_sum = seg_sum[:n_rows]
    if pad:
        seg_sum = jnp.pad(seg_sum, ((0, pad), (0, 0)))
        cnt = jnp.pad(cnt, ((0, pad), (0, 0)))
        res = jnp.pad(res, ((0, pad), (0, 0)))
    n_pad = n_rows + pad
    emb, res_out = pl.pallas_call(
        _finalize_body,
        grid=(n_pad // _BLK,),
        in_specs=[
            pl.BlockSpec((_BLK, D), lambda i: (i, 0)),
            pl.BlockSpec((_BLK, 1), lambda i: (i, 0)),
            pl.BlockSpec((_BLK, D), lambda i: (i, 0)),
        ],
        out_specs=[
            pl.BlockSpec((_BLK, D), lambda i: (i, 0)),
            pl.BlockSpec((_BLK, D), lambda i: (i, 0)),
        ],
        out_shape=[
            jax.ShapeDtypeStruct((n_pad, D), jnp.float32),
            jax.ShapeDtypeStruct((n_pad, D), jnp.float32),
        ],
    )(seg_sum, cnt, res)
    return emb[:n_rows], res_out[:n_rows]


def _user_finalize(user_agg, usr, intent, res):
    score = jax.nn.softmax(usr @ intent.T, axis=1)
    usr_agg = (score @ intent) * user_agg + user_agg
    ones = jnp.ones((usr_agg.shape[0], 1), jnp.float32)
    return _finalize(usr_agg, ones, res, usr_agg.shape[0])


def _pad_to(x, e_pad, fill):
    if x.shape[0] == e_pad:
        return x
    return jnp.concatenate([x, fill])


def _pad_edges(dst, src, w, n_dst_pad, n_src):
    e = dst.shape[0]
    step = NC * NS * KB
    e_pad = _cdiv(e, step) * step
    npad = e_pad - e
    ar = lax.iota(jnp.int32, npad)
    dstp = _pad_to(dst.astype(jnp.int32), e_pad,
                   n_dst_pad + (ar % (8 * L)))
    srcp = _pad_to(src.astype(jnp.int32), e_pad, ar % n_src)
    wp = None
    if w is not None:
        fill = (jnp.zeros((npad,), w.dtype) if w.dtype == jnp.float32
                else (ar % 32).astype(w.dtype))
        wp = _pad_to(w, e_pad, fill)
    return dstp, srcp, wp, e_pad


def kernel(user_emb, entity_emb, rel_weight, rel_intent_proj, hist_intent,
           ui_vals, edge_index, edge_type, ui_rows, ui_cols,
           adj_user_idx, adj_item_idx):
    relation_emb = rel_weight
    rel_int = rel_intent_proj @ relation_emb
    hist_int = hist_intent
    head, tail = edge_index[0], edge_index[1]

    n_ent_chunks = _cdiv(N_ENT, C)
    n_usr_chunks = _cdiv(N_USR, C)
    ent_pad = n_ent_chunks * C
    usr_pad = n_usr_chunks * C

    ui_r, ui_c, ui_w, e_ui = _pad_edges(ui_rows, ui_cols, ui_vals,
                                        usr_pad, N_ENT)
    adj_i, adj_u, _, e_adj = _pad_edges(adj_item_idx, adj_user_idx, None,
                                        ent_pad, N_USR)
    ehead, etail, etype, e_edge = _pad_edges(head, tail, edge_type,
                                             ent_pad, N_ENT)
    rel_flat = relation_emb.reshape((32 * D,))

    cnt2 = _build_counts(ent_pad, e_adj, e_edge)(adj_i, ehead)
    cnts = cnt2[0] + cnt2[1]
    cnt_adj = cnts[:N_ENT, None]
    cnt_edge = cnts[ent_pad:ent_pad + N_ENT, None]

    ui_k = _build_segsum(N_USR, e_ui, "scalar")
    adj_k = _build_segsum(N_ENT, e_adj, "plain")
    edge_k = _build_segsum(N_ENT, e_edge, "rel")

    def ui_agg(ent):
        return ui_k(ent, ui_r, ui_c, ui_w)[:N_USR]

    ui0 = ui_agg(entity_emb)

    adj_sum1 = adj_k(user_emb, adj_i, adj_u)
    h_ent, h_ent_res = _finalize(adj_sum1, cnt_adj, entity_emb, N_ENT)
    h_usr, h_usr_res = _user_finalize(ui0, user_emb, hist_int, user_emb)

    edge_sum1 = edge_k(entity_emb, ehead, etail, etype, rel_flat)
    r_ent, r_ent_res = _finalize(edge_sum1, cnt_edge, entity_emb, N_ENT)
    r_usr, r_usr_res = _user_finalize(ui0, user_emb, rel_int, user_emb)

    adj_sum2 = adj_k(h_usr, adj_i, adj_u)
    _, h_ent_res = _finalize(adj_sum2, cnt_adj, h_ent_res, N_ENT)
    _, h_usr_res = _user_finalize(ui_agg(h_ent), h_usr, hist_int, h_usr_res)

    edge_sum2 = edge_k(r_ent, ehead, etail, etype, rel_flat)
    _, r_ent_res = _finalize(edge_sum2, cnt_edge, r_ent_res, N_ENT)
    _, r_usr_res = _user_finalize(ui_agg(r_ent), r_usr, rel_int, r_usr_res)

    entity_res = jnp.concatenate([h_ent_res, r_ent_res], axis=-1)
    user_res = jnp.concatenate([h_usr_res, r_usr_res], axis=-1)
    return (entity_res, user_res, h_ent_res, r_ent_res, h_usr_res, r_usr_res)


# speedup vs baseline: 1.8741x; 1.0347x over previous
import functools

import jax
import jax.numpy as jnp
from jax import lax
from jax.experimental import pallas as pl
from jax.experimental.pallas import tpu as pltpu
from jax.experimental.pallas import tpu_sc as plsc

N_ENT = 100000
N_USR = 50000
N_FAC = 8
D = 128

NC = 2
NS = 16
L = 16

C = 12544
CT = C + 128
KB = 2048
ZR = 4
B = 32
NSLOT = 3

_CPARAMS = pltpu.CompilerParams(needs_layout_passes=False)
_MESH = dict(core_axis_name="c", subcore_axis_name="s")

_BLK = 1024


def _cdiv(a, b):
    return (a + b - 1) // b


@functools.lru_cache(maxsize=None)
def _build_segsum(n_dst, e_pad, mode):
    n_chunks = _cdiv(n_dst, C)
    out_rows = n_chunks * C
    P = _cdiv(n_chunks, NC)
    Et = e_pad // NS
    nblk = Et // KB
    assert Et % KB == 0
    slice_rows = CT // NS
    wrows = C // NS

    scratch = [
        pltpu.VMEM_SHARED((CT, D), jnp.float32),
        pltpu.VMEM((KB,), jnp.int32),
        pltpu.VMEM((KB,), jnp.int32),
        pltpu.VMEM((KB + 2 * L,), jnp.int32),
        pltpu.VMEM((KB + 2 * L,), jnp.int32),
        pltpu.VMEM((NSLOT * B, D), jnp.float32),
        pltpu.VMEM((ZR, D), jnp.float32),
        pltpu.SemaphoreType.DMA,
        pltpu.SemaphoreType.DMA,
    ]
    if mode == "scalar":
        scratch += [pltpu.VMEM((KB,), jnp.float32),
                    pltpu.VMEM((KB + 2 * L,), jnp.float32)]
    elif mode == "rel":
        scratch += [pltpu.VMEM((KB,), jnp.int32),
                    pltpu.VMEM((KB + 2 * L,), jnp.int32),
                    pltpu.VMEM((32 * D,), jnp.float32)]

    def body(*refs):
        if mode == "plain":
            (table, idx_dst_h, idx_src_h, out_h,
             spmem, dst_v, src_v, acc_dst, acc_src, rows, zrow,
             sem, sem2) = refs
            w_v = acc_w = rel_buf = None
        elif mode == "scalar":
            (table, idx_dst_h, idx_src_h, w_h, out_h,
             spmem, dst_v, src_v, acc_dst, acc_src, rows, zrow, sem, sem2,
             w_v, acc_w) = refs
            rel_buf = None
        else:
            (table, idx_dst_h, idx_src_h, w_h, rel_h, out_h,
             spmem, dst_v, src_v, acc_dst, acc_src, rows, zrow, sem, sem2,
             w_v, acc_w, rel_buf) = refs

        s = lax.axis_index("s")
        c = lax.axis_index("c")
        zero16 = jnp.zeros((L,), jnp.float32)

        for rz in range(ZR):
            for j in range(D // L):
                zrow[rz, pl.ds(j * L, L)] = zero16

        def zinit(i, carry):
            acc_src[pl.ds(i * L, L)] = jnp.zeros((L,), jnp.int32)
            return carry

        lax.fori_loop(0, (KB + 2 * L) // L, zinit, jnp.int32(0))
        if mode == "rel":
            pltpu.sync_copy(rel_h, rel_buf)

        def scale_batch(k, base):
            if mode == "plain":
                return
            for half in range(B // L):
                wv = acc_w[pl.ds(k * B + half * L, L)]
                for rr in range(L):
                    row = base + half * L + rr
                    if mode == "scalar":
                        w = wv[rr]
                        for j in range(D // L):
                            rows[row, pl.ds(j * L, L)] = (
                                rows[row, pl.ds(j * L, L)] * w)
                    else:
                        rb = wv[rr] * D
                        for j in range(D // L):
                            rows[row, pl.ds(j * L, L)] = (
                                rows[row, pl.ds(j * L, L)]
                                * rel_buf[pl.ds(rb + j * L, L)])

        def gfire(k):
            slot = lax.rem(k, NSLOT) * B
            pltpu.async_copy(table.at[acc_src.at[pl.ds(k * B, B)]],
                             rows.at[pl.ds(slot, B)], sem)

        def gwait(slot):
            pltpu.make_async_copy(table.at[pl.ds(0, B)],
                                  rows.at[pl.ds(slot, B)], sem).wait()

        def swait():
            pltpu.make_async_copy(rows.at[pl.ds(0, B)],
                                  spmem.at[pl.ds(0, B)], sem2).wait()

        def chunk_body(p, carry):
            chunk = p * NC + c

            @pl.when(chunk < n_chunks)
            def _():
                lo = chunk * C
                hi = lo + C
                zbase = s * slice_rows
                nzc = slice_rows // ZR
                for kk in range(nzc):
                    pltpu.async_copy(
                        zrow, spmem.at[pl.ds(zbase + kk * ZR, ZR)], sem)
                rem = slice_rows % ZR
                if rem:
                    pltpu.sync_copy(
                        zrow.at[pl.ds(0, rem)],
                        spmem.at[pl.ds(zbase + nzc * ZR, rem)])
                for kk in range(nzc):
                    pltpu.make_async_copy(
                        zrow, spmem.at[pl.ds(zbase, ZR)], sem).wait()
                plsc.subcore_barrier()

                def block_body(b, carry2):
                    base = s * Et + b * KB
                    pltpu.sync_copy(idx_dst_h.at[pl.ds(base, KB)], dst_v)
                    pltpu.sync_copy(idx_src_h.at[pl.ds(base, KB)], src_v)
                    if mode != "plain":
                        pltpu.sync_copy(w_h.at[pl.ds(base, KB)], w_v)

                    def cvec(k2, n):
                        dv = dst_v[pl.ds(k2 * L, L)]
                        m = (dv >= lo) & (dv < hi)
                        mi = m.astype(jnp.int32)
                        pos = n + plsc.cumsum(mi) - 1
                        plsc.store_scatter(acc_dst, [pos], dv - lo, mask=m)
                        sv = src_v[pl.ds(k2 * L, L)]
                        plsc.store_scatter(acc_src, [pos], sv, mask=m)
                        if mode != "plain":
                            wv = w_v[pl.ds(k2 * L, L)]
                            plsc.store_scatter(acc_w, [pos], wv, mask=m)
                        return n + jnp.sum(mi)

                    n = lax.fori_loop(0, KB // L, cvec, jnp.int32(0))

                    iota = lax.iota(jnp.int32, L)
                    dummy_src = iota * 16 + s * 37
                    for t in range(B // L):
                        acc_dst[pl.ds(n + t * L, L)] = (
                            jnp.int32(C + t * L) + iota)
                        acc_src[pl.ds(n + t * L, L)] = dummy_src
                        if mode != "plain":
                            acc_w[pl.ds(n + t * L, L)] = jnp.zeros(
                                (L,), acc_w.dtype)

                    nb = (n + B - 1) // B

                    @pl.when(nb >= 1)
                    def _p0():
                        gfire(jnp.int32(0))

                    @pl.when(nb >= 2)
                    def _p1():
                        gfire(jnp.int32(1))

                    def drain(k, carry3):
                        slot = lax.rem(k, NSLOT) * B
                        gwait(slot)
                        scale_batch(k, slot)
                        pltpu.async_copy(
                            rows.at[pl.ds(slot, B)],
                            spmem.at[acc_dst.at[pl.ds(k * B, B)]],
                            sem2, add=True)

                        @pl.when(k >= 1)
                        def _():
                            swait()

                        @pl.when(k + 2 < nb)
                        def _():
                            gfire(k + 2)

                        return carry3

                    lax.fori_loop(0, nb, drain, jnp.int32(0))

                    @pl.when(nb >= 1)
                    def _t0():
                        swait()

                    return carry2

                lax.fori_loop(0, nblk, block_body, jnp.int32(0))
                plsc.subcore_barrier()
                pltpu.sync_copy(
                    spmem.at[pl.ds(s * wrows, wrows)],
                    out_h.at[pl.ds(chunk * C + s * wrows, wrows)])
                plsc.subcore_barrier()

            return carry

        lax.fori_loop(0, P, chunk_body, jnp.int32(0))

    return pl.kernel(
        body,
        out_type=jax.ShapeDtypeStruct((out_rows, D), jnp.float32),
        mesh=plsc.VectorSubcoreMesh(**_MESH),
        scratch_types=scratch,
        compiler_params=_CPARAMS,
    )


@functools.lru_cache(maxsize=None)
def _build_counts(n_dst_pad, e1_pad, e2_pad):
    buf = n_dst_pad + 8 * L
    assert buf % (NS * 8) == 0
    per_tile = buf // NS
    wr = n_dst_pad // NS

    def body(idx1_h, idx2_h, out_h, cntA, cntB, idx2d, ones_v, zv,
             semL, semS):
        s = lax.axis_index("s")
        c = lax.axis_index("c")

        def fill(i, carry):
            zv[pl.ds(i * L, L)] = jnp.zeros((L,), jnp.float32)
            ones_v[pl.ds(i * L, L)] = jnp.ones((L,), jnp.float32)
            return carry

        lax.fori_loop(0, KB // L, fill, jnp.int32(0))

        zbase = s * per_tile
        nz = per_tile // KB
        for cnt in (cntA, cntB):
            for kk in range(nz):
                pltpu.async_copy(zv, cnt.at[pl.ds(zbase + kk * KB, KB)],
                                 semL)
            rem = per_tile % KB
            if rem:
                pltpu.sync_copy(zv.at[pl.ds(0, rem)],
                                cnt.at[pl.ds(zbase + nz * KB, rem)])
        for cnt in (cntA, cntB):
            for kk in range(nz):
                pltpu.make_async_copy(
                    zv, cnt.at[pl.ds(zbase, KB)], semL).wait()
        plsc.subcore_barrier()

        for idx_h, cnt, e_pad in ((idx1_h, cntA, e1_pad),
                                  (idx2_h, cntB, e2_pad)):
            Et = e_pad // (NC * NS)
            nblk = Et // KB
            assert Et % KB == 0
            tbase = (c * NS + s) * Et

            def lfire(b, idx_h=idx_h, tbase=tbase):
                slot = jnp.bitwise_and(b, 1) * KB
                pltpu.async_copy(idx_h.at[pl.ds(tbase + b * KB, KB)],
                                 idx2d.at[pl.ds(slot, KB)], semL)

            def lwait(idx_h=idx_h):
                pltpu.make_async_copy(idx_h.at[pl.ds(0, KB)],
                                      idx2d.at[pl.ds(0, KB)], semL).wait()

            def swaitc(cnt=cnt):
                pltpu.make_async_copy(ones_v, cnt.at[pl.ds(0, KB)],
                                      semS).wait()

            lfire(jnp.int32(0))

            def block_body(b, carry, cnt=cnt, lfire=lfire, lwait=lwait,
                           swaitc=swaitc):
                slot = jnp.bitwise_and(b, 1) * KB
                lwait()
                pltpu.async_copy(ones_v, cnt.at[idx2d.at[pl.ds(slot, KB)]],
                                 semS, add=True)

                @pl.when(b >= 1)
                def _():
                    swaitc()

                @pl.when(b + 1 < nblk)
                def _():
                    lfire(b + 1)

                return carry

            lax.fori_loop(0, nblk, block_body, jnp.int32(0))
            swaitc()
        plsc.subcore_barrier()
        pltpu.sync_copy(cntA.at[pl.ds(s * wr, wr)],
                        out_h.at[c, pl.ds(s * wr, wr)])
        pltpu.sync_copy(cntB.at[pl.ds(s * wr, wr)],
                        out_h.at[c, pl.ds(n_dst_pad + s * wr, wr)])

    return pl.kernel(
        body,
        out_type=jax.ShapeDtypeStruct((NC, 2 * n_dst_pad), jnp.float32),
        mesh=plsc.VectorSubcoreMesh(**_MESH),
        scratch_types=[
            pltpu.VMEM_SHARED((buf,), jnp.float32),
            pltpu.VMEM_SHARED((buf,), jnp.float32),
            pltpu.VMEM((2 * KB,), jnp.int32),
            pltpu.VMEM((KB,), jnp.float32),
            pltpu.VMEM((KB,), jnp.float32),
            pltpu.SemaphoreType.DMA,
            pltpu.SemaphoreType.DMA,
        ],
        compiler_params=_CPARAMS,
    )


def _finalize_body(sum_ref, cnt_ref, res_ref, emb_out_ref, res_out_ref):
    sval = sum_ref[...]
    cval = cnt_ref[...]
    mean = sval / jnp.maximum(cval, 1.0)
    nrm = jnp.sqrt(jnp.sum(mean * mean, axis=-1, keepdims=True))
    emb = mean / jnp.maximum(nrm, 1e-12)
    emb_out_ref[...] = emb
    res_out_ref[...] = res_ref[...] + emb


def _finalize(seg_sum, cnt, res, n_rows):
    pad = (-n_rows) % _BLK
    seg_sum = seg_sum[:n_rows]
    if pad:
        seg_sum = jnp.pad(seg_sum, ((0, pad), (0, 0)))
        cnt = jnp.pad(cnt, ((0, pad), (0, 0)))
        res = jnp.pad(res, ((0, pad), (0, 0)))
    n_pad = n_rows + pad
    emb, res_out = pl.pallas_call(
        _finalize_body,
        grid=(n_pad // _BLK,),
        in_specs=[
            pl.BlockSpec((_BLK, D), lambda i: (i, 0)),
            pl.BlockSpec((_BLK, 1), lambda i: (i, 0)),
            pl.BlockSpec((_BLK, D), lambda i: (i, 0)),
        ],
        out_specs=[
            pl.BlockSpec((_BLK, D), lambda i: (i, 0)),
            pl.BlockSpec((_BLK, D), lambda i: (i, 0)),
        ],
        out_shape=[
            jax.ShapeDtypeStruct((n_pad, D), jnp.float32),
            jax.ShapeDtypeStruct((n_pad, D), jnp.float32),
        ],
    )(seg_sum, cnt, res)
    return emb[:n_rows], res_out[:n_rows]


def _user_finalize(user_agg, usr, intent, res):
    score = jax.nn.softmax(usr @ intent.T, axis=1)
    usr_agg = (score @ intent) * user_agg + user_agg
    ones = jnp.ones((usr_agg.shape[0], 1), jnp.float32)
    return _finalize(usr_agg, ones, res, usr_agg.shape[0])


def _pad_to(x, e_pad, fill):
    if x.shape[0] == e_pad:
        return x
    return jnp.concatenate([x, fill])


def _pad_edges(dst, src, w, n_dst_pad, n_src):
    e = dst.shape[0]
    step = NC * NS * KB
    e_pad = _cdiv(e, step) * step
    npad = e_pad - e
    ar = lax.iota(jnp.int32, npad)
    dstp = _pad_to(dst.astype(jnp.int32), e_pad,
                   n_dst_pad + (ar % (8 * L)))
    srcp = _pad_to(src.astype(jnp.int32), e_pad, ar % n_src)
    wp = None
    if w is not None:
        fill = (jnp.zeros((npad,), w.dtype) if w.dtype == jnp.float32
                else (ar % 32).astype(w.dtype))
        wp = _pad_to(w, e_pad, fill)
    return dstp, srcp, wp, e_pad


def kernel(user_emb, entity_emb, rel_weight, rel_intent_proj, hist_intent,
           ui_vals, edge_index, edge_type, ui_rows, ui_cols,
           adj_user_idx, adj_item_idx):
    relation_emb = rel_weight
    rel_int = rel_intent_proj @ relation_emb
    hist_int = hist_intent
    head, tail = edge_index[0], edge_index[1]

    n_ent_chunks = _cdiv(N_ENT, C)
    n_usr_chunks = _cdiv(N_USR, C)
    ent_pad = n_ent_chunks * C
    usr_pad = n_usr_chunks * C

    ui_r, ui_c, ui_w, e_ui = _pad_edges(ui_rows, ui_cols, ui_vals,
                                        usr_pad, N_ENT)
    adj_i, adj_u, _, e_adj = _pad_edges(adj_item_idx, adj_user_idx, None,
                                        ent_pad, N_USR)
    ehead, etail, etype, e_edge = _pad_edges(head, tail, edge_type,
                                             ent_pad, N_ENT)
    rel_flat = relation_emb.reshape((32 * D,))

    cnt2 = _build_counts(ent_pad, e_adj, e_edge)(adj_i, ehead)
    cnts = cnt2[0] + cnt2[1]
    cnt_adj = cnts[:N_ENT, None]
    cnt_edge = cnts[ent_pad:ent_pad + N_ENT, None]

    ui_k = _build_segsum(N_USR, e_ui, "scalar")
    adj_k = _build_segsum(N_ENT, e_adj, "plain")
    edge_k = _build_segsum(N_ENT, e_edge, "rel")

    def ui_agg(ent):
        return ui_k(ent, ui_r, ui_c, ui_w)[:N_USR]

    ui0 = ui_agg(entity_emb)

    adj_sum1 = adj_k(user_emb, adj_i, adj_u)
    h_ent, h_ent_res = _finalize(adj_sum1, cnt_adj, entity_emb, N_ENT)
    h_usr, h_usr_res = _user_finalize(ui0, user_emb, hist_int, user_emb)

    edge_sum1 = edge_k(entity_emb, ehead, etail, etype, rel_flat)
    r_ent, r_ent_res = _finalize(edge_sum1, cnt_edge, entity_emb, N_ENT)
    r_usr, r_usr_res = _user_finalize(ui0, user_emb, rel_int, user_emb)

    adj_sum2 = adj_k(h_usr, adj_i, adj_u)
    _, h_ent_res = _finalize(adj_sum2, cnt_adj, h_ent_res, N_ENT)
    _, h_usr_res = _user_finalize(ui_agg(h_ent), h_usr, hist_int, h_usr_res)

    edge_sum2 = edge_k(r_ent, ehead, etail, etype, rel_flat)
    _, r_ent_res = _finalize(edge_sum2, cnt_edge, r_ent_res, N_ENT)
    _, r_usr_res = _user_finalize(ui_agg(r_ent), r_usr, rel_int, r_usr_res)

    entity_res = jnp.concatenate([h_ent_res, r_ent_res], axis=-1)
    user_res = jnp.concatenate([h_usr_res, r_usr_res], axis=-1)
    return (entity_res, user_res, h_ent_res, r_ent_res, h_usr_res, r_usr_res)
